# Initial kernel scaffold; baseline (speedup 1.0000x reference)
import jax
import jax.numpy as jnp
from jax.experimental import pallas as pl


def kernel(x, preds, labels, centers):
    raise NotImplementedError("write your pallas kernel here")


# speedup vs baseline: 1.5731x; 1.5731x over previous
import functools

import jax
import jax.numpy as jnp
from jax import lax
from jax.experimental import pallas as pl
from jax.experimental.pallas import tpu as pltpu
from jax.experimental.pallas import tpu_sc as plsc

NC = 2
NS = 16
L = 16
NW = NC * NS

_EPS = 1e-6
_NEG_INF = float("-inf")
AMAX_UNROLL = 8
DIST_UNROLL = 8


def _vsqrt(s):
    s = jnp.maximum(s, 1e-30)
    seed = (plsc.bitcast(s, jnp.int32) >> 1) + 0x1FBD1DF5
    y = plsc.bitcast(seed, jnp.float32)
    for _ in range(4):
        y = 0.5 * (y + s / y)
    return y


@functools.lru_cache(maxsize=None)
def _build_sc(B, C, D, V):
    blocks = B // (NW * L)

    @functools.partial(
        pl.kernel,
        out_type=jax.ShapeDtypeStruct((NW, L), jnp.float32),
        mesh=plsc.VectorSubcoreMesh(
            core_axis_name="c", subcore_axis_name="s",
            num_cores=NC, num_subcores=NS),
        compiler_params=pltpu.CompilerParams(
            use_tc_tiling_on_sc=False, needs_layout_passes=False),
        scratch_types=[
            pltpu.VMEM((2, L, C), jnp.float32),
            pltpu.VMEM((2, L, D), jnp.float32),
            pltpu.VMEM((2, L, D), jnp.float32),
            pltpu.VMEM((2, L, D), jnp.float32),
            pltpu.VMEM((blocks * L,), jnp.int32),
            pltpu.VMEM((2, L), jnp.int32),
            pltpu.VMEM((L,), jnp.float32),
        ] + [pltpu.SemaphoreType.DMA] * 9,
    )
    def sc_kernel(x_hbm, preds_hbm, labels_hbm, centers_hbm, out_hbm,
                  pblk, xblk, posb, negb, labv, advv, partv,
                  slab, sp0, sp1, sx0, sx1, spos0, spos1, sneg0, sneg1):
        wid = lax.axis_index("c") * NS + lax.axis_index("s")
        rows = lax.iota(jnp.int32, L)
        neg_inf_v = jnp.full((L,), _NEG_INF, jnp.float32)
        wbase = wid * (blocks * L)
        sp = (sp0, sp1)
        sx = (sx0, sx1)
        spos = (spos0, spos1)
        sneg = (sneg0, sneg1)

        def amax(buf, lab_vec):
            plsc.store_scatter(pblk.at[buf], [rows, lab_vec], neg_inf_v)

            def amax_body(j, carry):
                vmax, vidx = carry
                base = j * AMAX_UNROLL
                for k in range(AMAX_UNROLL):
                    col = jnp.full((L,), base + k, jnp.int32)
                    v = plsc.load_gather(pblk.at[buf], [rows, col])
                    upd = v > vmax
                    vmax = jnp.where(upd, v, vmax)
                    vidx = jnp.where(upd, col, vidx)
                return vmax, vidx

            _, vidx = lax.fori_loop(
                0, C // AMAX_UNROLL, amax_body,
                (neg_inf_v, jnp.zeros((L,), jnp.int32)))
            return vidx

        def dist(buf, part):
            skew = rows * 8

            def dist_body(j, carry):
                aap, aan = carry
                base = j * DIST_UNROLL
                for k in range(DIST_UNROLL):
                    col = (jnp.full((L,), base + k, jnp.int32) + skew) & (D - 1)
                    xv = plsc.load_gather(xblk.at[buf], [rows, col])
                    pv = plsc.load_gather(posb.at[buf], [rows, col])
                    nv = plsc.load_gather(negb.at[buf], [rows, col])
                    t = xv - pv + _EPS
                    u = xv - nv + _EPS
                    aap = aap + t * t
                    aan = aan + u * u
                return aap, aan

            aap, aan = lax.fori_loop(
                0, D // DIST_UNROLL, dist_body,
                (jnp.zeros((L,), jnp.float32), jnp.zeros((L,), jnp.float32)))
            return part + jnp.maximum(_vsqrt(aap) - _vsqrt(aan) + 1.0, 0.0)

        def start_preds(b):
            return pltpu.async_copy(
                preds_hbm.at[pl.ds(wbase + b * L, L), :], pblk.at[b % 2],
                sp[b % 2])

        def start_x_pos(b):
            cx = pltpu.async_copy(
                x_hbm.at[pl.ds(wbase + b * L, L), :], xblk.at[b % 2],
                sx[b % 2])
            cpos = pltpu.async_copy(
                centers_hbm.at[labv.at[pl.ds(b * L, L)]], posb.at[b % 2],
                spos[b % 2])
            return cx, cpos

        pltpu.sync_copy(labels_hbm.at[pl.ds(wbase, blocks * L)], labv)
        cp_preds = [None] * blocks
        cp_xpos = [None] * blocks
        cp_neg = [None] * blocks
        cp_preds[0] = start_preds(0)
        cp_xpos[0] = start_x_pos(0)

        part = jnp.zeros((L,), jnp.float32)
        for b in range(blocks):
            buf = b % 2
            if b + 1 < blocks:
                cp_preds[b + 1] = start_preds(b + 1)
            cp_preds[b].wait()
            lab_vec = labv[pl.ds(b * L, L)]
            vidx = amax(buf, lab_vec)
            advv[buf, :] = vidx
            cp_neg[b] = pltpu.async_copy(
                centers_hbm.at[advv.at[buf]], negb.at[buf], sneg[buf])
            if b > 0:
                cx, cpos = cp_xpos[b - 1]
                cx.wait()
                cpos.wait()
                cp_neg[b - 1].wait()
                part = dist((b - 1) % 2, part)
            if b + 1 < blocks:
                cp_xpos[b + 1] = start_x_pos(b + 1)
        cx, cpos = cp_xpos[blocks - 1]
        cx.wait()
        cpos.wait()
        cp_neg[blocks - 1].wait()
        part = dist((blocks - 1) % 2, part)

        partv[...] = part
        pltpu.sync_copy(partv, out_hbm.at[wid])

    return sc_kernel


@functools.lru_cache(maxsize=None)
def _build_finish(B):
    def body(p_ref, o_ref):
        o_ref[...] = jnp.sum(p_ref[...], keepdims=True) * (1.0 / B)

    return pl.pallas_call(
        body, out_shape=jax.ShapeDtypeStruct((1, 1), jnp.float32))


def kernel(x, preds, labels, centers):
    B, D = x.shape
    C = preds.shape[1]
    V = centers.shape[0]
    parts = _build_sc(B, C, D, V)(x, preds, labels.astype(jnp.int32), centers)
    return _build_finish(B)(parts)[0, 0]


# speedup vs baseline: 1.9073x; 1.2124x over previous
import functools

import jax
import jax.numpy as jnp
from jax import lax
from jax.experimental import pallas as pl
from jax.experimental.pallas import tpu as pltpu
from jax.experimental.pallas import tpu_sc as plsc

NC = 2
NS = 16
L = 16
NW = NC * NS

_EPS = 1e-6
_NEG_INF = float("-inf")
DIST_UNROLL = 8


def _vsqrt(s):
    s = jnp.maximum(s, 1e-30)
    seed = (plsc.bitcast(s, jnp.int32) >> 1) + 0x1FBD1DF5
    y = plsc.bitcast(seed, jnp.float32)
    for _ in range(4):
        y = 0.5 * (y + s / y)
    return y


@functools.lru_cache(maxsize=None)
def _build_amax(B, C):
    BLK = 256

    def body(p_ref, l_ref, o_ref):
        p = p_ref[...]
        lab = l_ref[...]
        cols = lax.broadcasted_iota(jnp.int32, (BLK, C), 1)
        masked = jnp.where(cols == lab[:, None], _NEG_INF, p)
        o_ref[...] = jnp.argmax(masked, axis=1).astype(jnp.int32)

    return pl.pallas_call(
        body,
        grid=(B // BLK,),
        in_specs=[pl.BlockSpec((BLK, C), lambda i: (i, 0)),
                  pl.BlockSpec((BLK,), lambda i: (i,))],
        out_specs=pl.BlockSpec((BLK,), lambda i: (i,)),
        out_shape=jax.ShapeDtypeStruct((B,), jnp.int32))


@functools.lru_cache(maxsize=None)
def _build_sc(B, D, V):
    blocks = B // (NW * L)

    @functools.partial(
        pl.kernel,
        out_type=jax.ShapeDtypeStruct((NW, L), jnp.float32),
        mesh=plsc.VectorSubcoreMesh(
            core_axis_name="c", subcore_axis_name="s",
            num_cores=NC, num_subcores=NS),
        compiler_params=pltpu.CompilerParams(
            use_tc_tiling_on_sc=False, needs_layout_passes=False),
        scratch_types=[
            pltpu.VMEM((2, L, D), jnp.float32),
            pltpu.VMEM((2, L, D), jnp.float32),
            pltpu.VMEM((2, L, D), jnp.float32),
            pltpu.VMEM((blocks * L,), jnp.int32),
            pltpu.VMEM((blocks * L,), jnp.int32),
            pltpu.VMEM((L,), jnp.float32),
        ] + [pltpu.SemaphoreType.DMA] * 8,
    )
    def sc_kernel(x_hbm, labels_hbm, adv_hbm, centers_hbm, out_hbm,
                  xblk, posb, negb, labv, advv, partv,
                  slab, sadv, sx0, sx1, spos0, spos1, sneg0, sneg1):
        wid = lax.axis_index("c") * NS + lax.axis_index("s")
        rows = lax.iota(jnp.int32, L)
        wbase = wid * (blocks * L)
        sx = (sx0, sx1)
        spos = (spos0, spos1)
        sneg = (sneg0, sneg1)

        def dist(buf, part):
            skew = rows * 8

            def dist_body(j, carry):
                aap, aan = carry
                base = j * DIST_UNROLL
                for k in range(DIST_UNROLL):
                    col = (jnp.full((L,), base + k, jnp.int32) + skew) & (D - 1)
                    xv = plsc.load_gather(xblk.at[buf], [rows, col])
                    pv = plsc.load_gather(posb.at[buf], [rows, col])
                    nv = plsc.load_gather(negb.at[buf], [rows, col])
                    t = xv - pv + _EPS
                    u = xv - nv + _EPS
                    aap = aap + t * t
                    aan = aan + u * u
                return aap, aan

            aap, aan = lax.fori_loop(
                0, D // DIST_UNROLL, dist_body,
                (jnp.zeros((L,), jnp.float32), jnp.zeros((L,), jnp.float32)))
            return part + jnp.maximum(_vsqrt(aap) - _vsqrt(aan) + 1.0, 0.0)

        def start_block(b):
            buf = b % 2
            cx = pltpu.async_copy(
                x_hbm.at[pl.ds(wbase + b * L, L), :], xblk.at[buf], sx[buf])
            cpos = pltpu.async_copy(
                centers_hbm.at[labv.at[pl.ds(b * L, L)]], posb.at[buf],
                spos[buf])
            cneg = pltpu.async_copy(
                centers_hbm.at[advv.at[pl.ds(b * L, L)]], negb.at[buf],
                sneg[buf])
            return cx, cpos, cneg

        cl = pltpu.async_copy(labels_hbm.at[pl.ds(wbase, blocks * L)], labv,
                              slab)
        ca = pltpu.async_copy(adv_hbm.at[pl.ds(wbase, blocks * L)], advv,
                              sadv)
        cl.wait()
        ca.wait()
        cps = [None] * blocks
        cps[0] = start_block(0)

        part = jnp.zeros((L,), jnp.float32)
        for b in range(blocks):
            if b + 1 < blocks:
                cps[b + 1] = start_block(b + 1)
            for c in cps[b]:
                c.wait()
            part = dist(b % 2, part)

        partv[...] = part
        pltpu.sync_copy(partv, out_hbm.at[wid])

    return sc_kernel


@functools.lru_cache(maxsize=None)
def _build_finish(B):
    def body(p_ref, o_ref):
        o_ref[...] = jnp.sum(p_ref[...], keepdims=True) * (1.0 / B)

    return pl.pallas_call(
        body, out_shape=jax.ShapeDtypeStruct((1, 1), jnp.float32))


def kernel(x, preds, labels, centers):
    B, D = x.shape
    C = preds.shape[1]
    V = centers.shape[0]
    labels = labels.astype(jnp.int32)
    adv = _build_amax(B, C)(preds, labels)
    parts = _build_sc(B, D, V)(x, labels, adv, centers)
    return _build_finish(B)(parts)[0, 0]


# speedup vs baseline: 1.9363x; 1.0152x over previous
import functools

import jax
import jax.numpy as jnp
from jax import lax
from jax.experimental import pallas as pl
from jax.experimental.pallas import tpu as pltpu
from jax.experimental.pallas import tpu_sc as plsc

NC = 2
NS = 16
L = 16
NW = NC * NS

_EPS = 1e-6
_NEG_INF = float("-inf")
DIST_UNROLL = 8
NBUF = 3


def _vsqrt(s):
    s = jnp.maximum(s, 1e-30)
    seed = (plsc.bitcast(s, jnp.int32) >> 1) + 0x1FBD1DF5
    y = plsc.bitcast(seed, jnp.float32)
    for _ in range(4):
        y = 0.5 * (y + s / y)
    return y


@functools.lru_cache(maxsize=None)
def _build_amax(B, C):
    BLK = 256

    def body(p_ref, l_ref, o_ref):
        p = p_ref[...]
        lab = l_ref[...]
        cols = lax.broadcasted_iota(jnp.int32, (BLK, C), 1)
        keep = cols != lab[:, None]
        masked = jnp.where(keep, p, _NEG_INF)
        m = jnp.max(masked, axis=1, keepdims=True)
        hit = jnp.where(keep & (masked == m), cols, C)
        o_ref[...] = jnp.min(hit, axis=1).astype(jnp.int32)

    return pl.pallas_call(
        body,
        grid=(B // BLK,),
        in_specs=[pl.BlockSpec((BLK, C), lambda i: (i, 0)),
                  pl.BlockSpec((BLK,), lambda i: (i,))],
        out_specs=pl.BlockSpec((BLK,), lambda i: (i,)),
        out_shape=jax.ShapeDtypeStruct((B,), jnp.int32))


@functools.lru_cache(maxsize=None)
def _build_sc(B, D, V):
    blocks = B // (NW * L)

    @functools.partial(
        pl.kernel,
        out_type=jax.ShapeDtypeStruct((NW * L,), jnp.float32),
        mesh=plsc.VectorSubcoreMesh(
            core_axis_name="c", subcore_axis_name="s",
            num_cores=NC, num_subcores=NS),
        compiler_params=pltpu.CompilerParams(
            use_tc_tiling_on_sc=False, needs_layout_passes=False),
        scratch_types=[
            pltpu.VMEM((NBUF, L, D), jnp.float32),
            pltpu.VMEM((NBUF, L, D), jnp.float32),
            pltpu.VMEM((NBUF, L, D), jnp.float32),
            pltpu.VMEM((blocks * L,), jnp.int32),
            pltpu.VMEM((blocks * L,), jnp.int32),
            pltpu.VMEM((L,), jnp.float32),
        ] + [pltpu.SemaphoreType.DMA] * (3 * NBUF + 2),
    )
    def sc_kernel(x_hbm, labels_hbm, adv_hbm, centers_hbm, out_hbm,
                  xblk, posb, negb, labv, advv, partv, *sems):
        wid = lax.axis_index("c") * NS + lax.axis_index("s")
        rows = lax.iota(jnp.int32, L)
        wbase = wid * (blocks * L)
        sx = sems[0:NBUF]
        spos = sems[NBUF:2 * NBUF]
        sneg = sems[2 * NBUF:3 * NBUF]
        slab, sadv = sems[3 * NBUF], sems[3 * NBUF + 1]

        def dist(buf, part):
            skew = rows * 8

            def dist_body(j, carry):
                aap, aan = carry
                base = j * DIST_UNROLL
                for k in range(DIST_UNROLL):
                    col = (jnp.full((L,), base + k, jnp.int32) + skew) & (D - 1)
                    xv = plsc.load_gather(xblk.at[buf], [rows, col])
                    pv = plsc.load_gather(posb.at[buf], [rows, col])
                    nv = plsc.load_gather(negb.at[buf], [rows, col])
                    t = xv - pv + _EPS
                    u = xv - nv + _EPS
                    aap = aap + t * t
                    aan = aan + u * u
                return aap, aan

            aap, aan = lax.fori_loop(
                0, D // DIST_UNROLL, dist_body,
                (jnp.zeros((L,), jnp.float32), jnp.zeros((L,), jnp.float32)))
            return part + jnp.maximum(_vsqrt(aap) - _vsqrt(aan) + 1.0, 0.0)

        def start_block(b):
            buf = b % NBUF
            cx = pltpu.async_copy(
                x_hbm.at[pl.ds(wbase + b * L, L), :], xblk.at[buf], sx[buf])
            cpos = pltpu.async_copy(
                centers_hbm.at[labv.at[pl.ds(b * L, L)]], posb.at[buf],
                spos[buf])
            cneg = pltpu.async_copy(
                centers_hbm.at[advv.at[pl.ds(b * L, L)]], negb.at[buf],
                sneg[buf])
            return cx, cpos, cneg

        cl = pltpu.async_copy(labels_hbm.at[pl.ds(wbase, blocks * L)], labv,
                              slab)
        ca = pltpu.async_copy(adv_hbm.at[pl.ds(wbase, blocks * L)], advv,
                              sadv)
        cl.wait()
        ca.wait()
        cps = [None] * blocks
        for b in range(NBUF - 1):
            cps[b] = start_block(b)

        part = jnp.zeros((L,), jnp.float32)
        for b in range(blocks):
            if b + NBUF - 1 < blocks:
                cps[b + NBUF - 1] = start_block(b + NBUF - 1)
            for c in cps[b]:
                c.wait()
            part = dist(b % NBUF, part)

        partv[...] = part
        pltpu.sync_copy(partv, out_hbm.at[pl.ds(wid * L, L)])

    return sc_kernel


@functools.lru_cache(maxsize=None)
def _build_finish(B, P):
    def body(p_ref, o_ref):
        o_ref[...] = jnp.sum(p_ref[...], keepdims=True) * (1.0 / B)

    return pl.pallas_call(
        body, out_shape=jax.ShapeDtypeStruct((1,), jnp.float32))


def kernel(x, preds, labels, centers):
    B, D = x.shape
    C = preds.shape[1]
    V = centers.shape[0]
    labels = labels.astype(jnp.int32)
    adv = _build_amax(B, C)(preds, labels)
    parts = _build_sc(B, D, V)(x, labels, adv, centers)
    return _build_finish(B, NW * L)(parts)[0]
